# Initial kernel scaffold; baseline (speedup 1.0000x reference)
import jax
import jax.numpy as jnp
from jax.experimental import pallas as pl


def kernel(x, W1, W2, W3, W4, W5):
    raise NotImplementedError("write your pallas kernel here")


# speedup vs baseline: 16.6054x; 16.6054x over previous
import functools

import jax
import jax.numpy as jnp
from jax import lax
from jax.experimental import pallas as pl
from jax.experimental.pallas import tpu as pltpu
from jax.experimental.pallas import tpu_sc as plsc

K = 4


def _tc_layer_body(n, x_ref, wn_ref, wc_ref, idx_ref, t_ref, ctr_ref):
    b = pl.program_id(0)
    xb = x_ref[...]
    sqv = jnp.sum(xb * xb, axis=1)
    g = lax.dot_general(xb, xb, (((1,), (1,)), ((), ())),
                        preferred_element_type=jnp.float32)
    d = 2.0 * g - sqv[None, :]
    iota = lax.broadcasted_iota(jnp.int32, (n, n), 1)
    for k in range(K):
        m = jnp.max(d, axis=1, keepdims=True)
        cand = jnp.where(d == m, iota, n)
        ik = jnp.min(cand, axis=1, keepdims=True)
        idx_ref[:, k:k + 1] = ik + b * n
        if k < K - 1:
            d = jnp.where(iota == ik, -jnp.inf, d)
    t_ref[...] = jnp.dot(xb, wn_ref[...], preferred_element_type=jnp.float32)
    ctr_ref[...] = jnp.dot(xb, wc_ref[...], preferred_element_type=jnp.float32)


def _tc_layer(xp, wn_t, wc_t, bsz, n):
    c = xp.shape[1]
    o = wn_t.shape[1]
    return pl.pallas_call(
        functools.partial(_tc_layer_body, n),
        grid=(bsz,),
        in_specs=[
            pl.BlockSpec((n, c), lambda b: (b, 0)),
            pl.BlockSpec((c, o), lambda b: (0, 0)),
            pl.BlockSpec((c, o), lambda b: (0, 0)),
        ],
        out_specs=[
            pl.BlockSpec((n, K), lambda b: (b, 0)),
            pl.BlockSpec((n, o), lambda b: (b, 0)),
            pl.BlockSpec((n, o), lambda b: (b, 0)),
        ],
        out_shape=[
            jax.ShapeDtypeStruct((bsz * n, K), jnp.int32),
            jax.ShapeDtypeStruct((bsz * n, o), jnp.float32),
            jax.ShapeDtypeStruct((bsz * n, o), jnp.float32),
        ],
    )(xp, wn_t, wc_t)


def _sc_layer(t, ctr, idx_flat):
    bn, o = t.shape
    nw = 32
    rows_w = bn // nw
    ch = 32
    nch = rows_w // ch
    mesh = plsc.VectorSubcoreMesh(core_axis_name="c", subcore_axis_name="s")

    @functools.partial(
        pl.kernel,
        mesh=mesh,
        out_type=jax.ShapeDtypeStruct((bn, o), jnp.float32),
        scratch_types=[
            pltpu.VMEM((rows_w * K,), jnp.int32),
            pltpu.VMEM((ch * K, o), jnp.float32),
            pltpu.VMEM((ch, o), jnp.float32),
            pltpu.VMEM((ch, o), jnp.float32),
            pltpu.SemaphoreType.DMA,
        ],
    )
    def sc_k(t_hbm, ctr_hbm, idx_hbm, out_hbm, idx_v, rows_v, ctr_v, out_v,
             sem):
        wid = lax.axis_index("s") * 2 + lax.axis_index("c")
        base = wid * rows_w
        pltpu.sync_copy(idx_hbm.at[pl.ds(base * K, rows_w * K)], idx_v)
        for ci in range(nch):
            r0 = base + ci * ch
            gather = pltpu.async_copy(
                t_hbm.at[idx_v.at[pl.ds(ci * ch * K, ch * K)]], rows_v, sem)
            pltpu.sync_copy(ctr_hbm.at[pl.ds(r0, ch)], ctr_v)
            gather.wait()

            def body(p, carry):
                for j in range(o // 16):
                    s = pl.ds(j * 16, 16)
                    v = jnp.maximum(
                        jnp.maximum(rows_v[K * p, s], rows_v[K * p + 1, s]),
                        jnp.maximum(rows_v[K * p + 2, s],
                                    rows_v[K * p + 3, s]))
                    out_v[p, s] = jnp.maximum(v + ctr_v[p, s], 0.0)
                return carry

            lax.fori_loop(0, ch, body, 0)
            pltpu.sync_copy(out_v, out_hbm.at[pl.ds(r0, ch)])

    return sc_k(t, ctr, idx_flat)


def _tc_final_body(x1_ref, x2_ref, x3_ref, x4_ref, w5_ref, out_ref):
    cat = jnp.concatenate(
        [x1_ref[...][:, :64], x2_ref[...][:, :64], x3_ref[...], x4_ref[...]],
        axis=1)
    r = lax.dot_general(w5_ref[...], cat, (((1,), (1,)), ((), ())),
                        preferred_element_type=jnp.float32)
    out_ref[...] = jnp.tanh(r)[None]


def _tc_final(feats, w5, bsz, n):
    x1, x2, x3, x4 = feats
    specs = [pl.BlockSpec((n, f.shape[1]), lambda b: (b, 0)) for f in feats]
    return pl.pallas_call(
        _tc_final_body,
        grid=(bsz,),
        in_specs=specs + [pl.BlockSpec((512, 512), lambda b: (0, 0))],
        out_specs=pl.BlockSpec((1, 512, n), lambda b: (b, 0, 0)),
        out_shape=jax.ShapeDtypeStruct((bsz, 512, n), jnp.float32),
    )(x1, x2, x3, x4, w5)


def kernel(x, W1, W2, W3, W4, W5):
    bsz, c0, n = x.shape
    cur = jnp.transpose(x, (0, 2, 1)).reshape(bsz * n, c0)
    feats = []
    for w, cin in ((W1, c0), (W2, 64), (W3, 64), (W4, 128)):
        o = w.shape[0]
        op = max(o, 128)
        cp = cur.shape[1]
        wn_t = jnp.zeros((cp, op), jnp.float32).at[:cin, :o].set(
            jnp.transpose(w[:, :cin]))
        wc_t = jnp.zeros((cp, op), jnp.float32).at[:cin, :o].set(
            jnp.transpose(w[:, cin:]))
        idx, t, ctr = _tc_layer(cur, wn_t, wc_t, bsz, n)
        cur = _sc_layer(t, ctr, idx.reshape(-1))
        feats.append(cur)
    return _tc_final(feats, W5, bsz, n)


# speedup vs baseline: 21.3573x; 1.2862x over previous
import functools

import jax
import jax.numpy as jnp
from jax import lax
from jax.experimental import pallas as pl
from jax.experimental.pallas import tpu as pltpu
from jax.experimental.pallas import tpu_sc as plsc

K = 4


def _tc_layer_body(n, x_ref, wn_ref, wc_ref, idx_ref, t_ref, ctr_ref):
    b = pl.program_id(0)
    xb = x_ref[...]
    sqv = jnp.sum(xb * xb, axis=1)
    g = lax.dot_general(xb, xb, (((1,), (1,)), ((), ())),
                        preferred_element_type=jnp.float32)
    d = 2.0 * g - sqv[None, :]
    iota = lax.broadcasted_iota(jnp.int32, (n, n), 1)
    for k in range(K):
        m = jnp.max(d, axis=1, keepdims=True)
        cand = jnp.where(d == m, iota, n)
        ik = jnp.min(cand, axis=1, keepdims=True)
        idx_ref[:, k:k + 1] = ik + b * n
        if k < K - 1:
            d = jnp.where(iota == ik, -jnp.inf, d)
    t_ref[...] = jnp.dot(xb, wn_ref[...], preferred_element_type=jnp.float32)
    ctr_ref[...] = jnp.dot(xb, wc_ref[...], preferred_element_type=jnp.float32)


def _tc_layer(xp, wn_t, wc_t, bsz, n):
    c = xp.shape[1]
    o = wn_t.shape[1]
    return pl.pallas_call(
        functools.partial(_tc_layer_body, n),
        grid=(bsz,),
        in_specs=[
            pl.BlockSpec((n, c), lambda b: (b, 0)),
            pl.BlockSpec((c, o), lambda b: (0, 0)),
            pl.BlockSpec((c, o), lambda b: (0, 0)),
        ],
        out_specs=[
            pl.BlockSpec((n, K), lambda b: (b, 0)),
            pl.BlockSpec((n, o), lambda b: (b, 0)),
            pl.BlockSpec((n, o), lambda b: (b, 0)),
        ],
        out_shape=[
            jax.ShapeDtypeStruct((bsz * n, K), jnp.int32),
            jax.ShapeDtypeStruct((bsz * n, o), jnp.float32),
            jax.ShapeDtypeStruct((bsz * n, o), jnp.float32),
        ],
    )(xp, wn_t, wc_t)


def _sc_layer(t, ctr, idx_flat):
    bn, o = t.shape
    nw = 32
    rows_w = bn // nw
    ch = 32
    nch = rows_w // ch
    mesh = plsc.VectorSubcoreMesh(core_axis_name="c", subcore_axis_name="s")

    @functools.partial(
        pl.kernel,
        mesh=mesh,
        out_type=jax.ShapeDtypeStruct((bn, o), jnp.float32),
        scratch_types=[
            pltpu.VMEM((rows_w * K,), jnp.int32),
            pltpu.VMEM((ch * K, o), jnp.float32),
            pltpu.VMEM((ch, o), jnp.float32),
            pltpu.VMEM((ch, o), jnp.float32),
            pltpu.SemaphoreType.DMA,
        ],
    )
    def sc_k(t_hbm, ctr_hbm, idx_hbm, out_hbm, idx_v, rows_v, ctr_v, out_v,
             sem):
        wid = lax.axis_index("s") * 2 + lax.axis_index("c")
        base = wid * rows_w
        pltpu.sync_copy(idx_hbm.at[pl.ds(base * K, rows_w * K)], idx_v)
        for ci in range(nch):
            r0 = base + ci * ch
            gather = pltpu.async_copy(
                t_hbm.at[idx_v.at[pl.ds(ci * ch * K, ch * K)]], rows_v, sem)
            pltpu.sync_copy(ctr_hbm.at[pl.ds(r0, ch)], ctr_v)
            gather.wait()

            def body(p, carry):
                for j in range(o // 16):
                    s = pl.ds(j * 16, 16)
                    v = jnp.maximum(
                        jnp.maximum(rows_v[K * p, s], rows_v[K * p + 1, s]),
                        jnp.maximum(rows_v[K * p + 2, s],
                                    rows_v[K * p + 3, s]))
                    out_v[p, s] = jnp.maximum(v + ctr_v[p, s], 0.0)
                return carry

            lax.fori_loop(0, ch, body, 0)
            pltpu.sync_copy(out_v, out_hbm.at[pl.ds(r0, ch)])

    return sc_k(t, ctr, idx_flat)


def _tc_final_body(x1_ref, x2_ref, x3_ref, x4_ref, w5_ref, out_ref):
    cat = jnp.concatenate(
        [x1_ref[...][:, :64], x2_ref[...][:, :64], x3_ref[...], x4_ref[...]],
        axis=1)
    r = lax.dot_general(w5_ref[...], cat, (((1,), (1,)), ((), ())),
                        preferred_element_type=jnp.float32)
    out_ref[...] = jnp.tanh(r)[None]


def _tc_final(feats, w5, bsz, n):
    x1, x2, x3, x4 = feats
    specs = [pl.BlockSpec((n, f.shape[1]), lambda b: (b, 0)) for f in feats]
    return pl.pallas_call(
        _tc_final_body,
        grid=(bsz,),
        in_specs=specs + [pl.BlockSpec((512, 512), lambda b: (0, 0))],
        out_specs=pl.BlockSpec((1, 512, n), lambda b: (b, 0, 0)),
        out_shape=jax.ShapeDtypeStruct((bsz, 512, n), jnp.float32),
    )(x1, x2, x3, x4, w5)


def kernel(x, W1, W2, W3, W4, W5):
    bsz, c0, n = x.shape
    xt = jnp.transpose(x, (0, 2, 1)).reshape(bsz * n, c0)
    weights = ((W1, c0), (W2, 64), (W3, 64), (W4, 128))
    hb = bsz // 2
    outs = []
    for h in range(2):
        cur = lax.slice_in_dim(xt, h * hb * n, (h + 1) * hb * n, axis=0)
        feats = []
        for w, cin in weights:
            o = w.shape[0]
            op = max(o, 128)
            cp = cur.shape[1]
            wn_t = jnp.zeros((cp, op), jnp.float32).at[:cin, :o].set(
                jnp.transpose(w[:, :cin]))
            wc_t = jnp.zeros((cp, op), jnp.float32).at[:cin, :o].set(
                jnp.transpose(w[:, cin:]))
            idx, t, ctr = _tc_layer(cur, wn_t, wc_t, hb, n)
            cur = _sc_layer(t, ctr, idx.reshape(-1))
            feats.append(cur)
        outs.append(_tc_final(feats, W5, hb, n))
    return jnp.concatenate(outs, axis=0)


# speedup vs baseline: 23.5097x; 1.1008x over previous
import functools

import jax
import jax.numpy as jnp
from jax import lax
from jax.experimental import pallas as pl
from jax.experimental.pallas import tpu as pltpu
from jax.experimental.pallas import tpu_sc as plsc

K = 4


def _tc_layer_body(n, x_ref, wn_ref, wc_ref, idx_ref, t_ref, ctr_ref):
    b = pl.program_id(0)
    xb = x_ref[...]
    sqv = jnp.sum(xb * xb, axis=1)
    g = lax.dot_general(xb, xb, (((1,), (1,)), ((), ())),
                        preferred_element_type=jnp.float32)
    d = 2.0 * g - sqv[None, :]
    iotaf = lax.broadcasted_iota(jnp.int32, (n, n), 1).astype(jnp.float32)
    for k in range(K):
        m = jnp.max(d, axis=1, keepdims=True)
        cand = jnp.where(d == m, iotaf, jnp.float32(n))
        ikf = jnp.min(cand, axis=1, keepdims=True)
        idx_ref[:, k:k + 1] = ikf.astype(jnp.int32) + b * n
        if k < K - 1:
            d = jnp.where(iotaf == ikf, -jnp.inf, d)
    t_ref[...] = jnp.dot(xb, wn_ref[...], preferred_element_type=jnp.float32)
    ctr_ref[...] = jnp.dot(xb, wc_ref[...], preferred_element_type=jnp.float32)


def _tc_layer(xp, wn_t, wc_t, bsz, n):
    c = xp.shape[1]
    o = wn_t.shape[1]
    return pl.pallas_call(
        functools.partial(_tc_layer_body, n),
        grid=(bsz,),
        in_specs=[
            pl.BlockSpec((n, c), lambda b: (b, 0)),
            pl.BlockSpec((c, o), lambda b: (0, 0)),
            pl.BlockSpec((c, o), lambda b: (0, 0)),
        ],
        out_specs=[
            pl.BlockSpec((n, K), lambda b: (b, 0)),
            pl.BlockSpec((n, o), lambda b: (b, 0)),
            pl.BlockSpec((n, o), lambda b: (b, 0)),
        ],
        out_shape=[
            jax.ShapeDtypeStruct((bsz * n, K), jnp.int32),
            jax.ShapeDtypeStruct((bsz * n, o), jnp.float32),
            jax.ShapeDtypeStruct((bsz * n, o), jnp.float32),
        ],
    )(xp, wn_t, wc_t)


def _sc_layer(t, ctr, idx_flat):
    bn, o = t.shape
    nw = 32
    rows_w = bn // nw
    ch = 32
    nch = rows_w // ch
    mesh = plsc.VectorSubcoreMesh(core_axis_name="c", subcore_axis_name="s")

    @functools.partial(
        pl.kernel,
        mesh=mesh,
        out_type=jax.ShapeDtypeStruct((bn, o), jnp.float32),
        scratch_types=[
            pltpu.VMEM((rows_w * K,), jnp.int32),
            pltpu.VMEM((ch * K, o), jnp.float32),
            pltpu.VMEM((ch, o), jnp.float32),
            pltpu.VMEM((ch, o), jnp.float32),
            pltpu.SemaphoreType.DMA,
        ],
    )
    def sc_k(t_hbm, ctr_hbm, idx_hbm, out_hbm, idx_v, rows_v, ctr_v, out_v,
             sem):
        wid = lax.axis_index("s") * 2 + lax.axis_index("c")
        base = wid * rows_w
        pltpu.sync_copy(idx_hbm.at[pl.ds(base * K, rows_w * K)], idx_v)
        for ci in range(nch):
            r0 = base + ci * ch
            gather = pltpu.async_copy(
                t_hbm.at[idx_v.at[pl.ds(ci * ch * K, ch * K)]], rows_v, sem)
            pltpu.sync_copy(ctr_hbm.at[pl.ds(r0, ch)], ctr_v)
            gather.wait()

            def body(p, carry):
                for j in range(o // 16):
                    s = pl.ds(j * 16, 16)
                    v = jnp.maximum(
                        jnp.maximum(rows_v[K * p, s], rows_v[K * p + 1, s]),
                        jnp.maximum(rows_v[K * p + 2, s],
                                    rows_v[K * p + 3, s]))
                    out_v[p, s] = jnp.maximum(v + ctr_v[p, s], 0.0)
                return carry

            lax.fori_loop(0, ch, body, 0)
            pltpu.sync_copy(out_v, out_hbm.at[pl.ds(r0, ch)])

    return sc_k(t, ctr, idx_flat)


def _tc_final_body(x1_ref, x2_ref, x3_ref, x4_ref, w5_ref, out_ref):
    cat = jnp.concatenate(
        [x1_ref[...][:, :64], x2_ref[...][:, :64], x3_ref[...], x4_ref[...]],
        axis=1)
    r = lax.dot_general(w5_ref[...], cat, (((1,), (1,)), ((), ())),
                        preferred_element_type=jnp.float32)
    out_ref[...] = jnp.tanh(r)[None]


def _tc_final(feats, w5, bsz, n):
    x1, x2, x3, x4 = feats
    specs = [pl.BlockSpec((n, f.shape[1]), lambda b: (b, 0)) for f in feats]
    return pl.pallas_call(
        _tc_final_body,
        grid=(bsz,),
        in_specs=specs + [pl.BlockSpec((512, 512), lambda b: (0, 0))],
        out_specs=pl.BlockSpec((1, 512, n), lambda b: (b, 0, 0)),
        out_shape=jax.ShapeDtypeStruct((bsz, 512, n), jnp.float32),
    )(x1, x2, x3, x4, w5)


def kernel(x, W1, W2, W3, W4, W5):
    bsz, c0, n = x.shape
    xt = jnp.transpose(x, (0, 2, 1)).reshape(bsz * n, c0)
    weights = ((W1, c0), (W2, 64), (W3, 64), (W4, 128))
    hb = bsz // 2
    outs = []
    for h in range(2):
        cur = lax.slice_in_dim(xt, h * hb * n, (h + 1) * hb * n, axis=0)
        feats = []
        for w, cin in weights:
            o = w.shape[0]
            op = max(o, 128)
            cp = cur.shape[1]
            wn_t = jnp.zeros((cp, op), jnp.float32).at[:cin, :o].set(
                jnp.transpose(w[:, :cin]))
            wc_t = jnp.zeros((cp, op), jnp.float32).at[:cin, :o].set(
                jnp.transpose(w[:, cin:]))
            idx, t, ctr = _tc_layer(cur, wn_t, wc_t, hb, n)
            cur = _sc_layer(t, ctr, idx.reshape(-1))
            feats.append(cur)
        outs.append(_tc_final(feats, W5, hb, n))
    return jnp.concatenate(outs, axis=0)


# speedup vs baseline: 26.6409x; 1.1332x over previous
import functools

import jax
import jax.numpy as jnp
from jax import lax
from jax.experimental import pallas as pl
from jax.experimental.pallas import tpu as pltpu
from jax.experimental.pallas import tpu_sc as plsc

K = 4


def _tc_layer_body(n, x_ref, wn_ref, wc_ref, idx_ref, t_ref, ctr_ref):
    b = pl.program_id(0)
    xb = x_ref[...]
    sqc = jnp.sum(xb * xb, axis=1, keepdims=True)
    g = lax.dot_general(xb, xb, (((1,), (1,)), ((), ())),
                        preferred_element_type=jnp.float32)
    d = 2.0 * g - sqc
    iotaf = lax.broadcasted_iota(jnp.int32, (n, n), 0).astype(jnp.float32)
    for k in range(K):
        m = jnp.max(d, axis=0, keepdims=True)
        cand = jnp.where(d == m, iotaf, jnp.float32(n))
        ikf = jnp.min(cand, axis=0, keepdims=True)
        idx_ref[k:k + 1, :] = ikf.astype(jnp.int32) + b * n
        if k < K - 1:
            d = jnp.where(iotaf == ikf, -jnp.inf, d)
    idx_ref[K:, :] = jnp.zeros((8 - K, n), jnp.int32)
    t_ref[...] = jnp.dot(xb, wn_ref[...], preferred_element_type=jnp.float32)
    ctr_ref[...] = jnp.dot(xb, wc_ref[...], preferred_element_type=jnp.float32)


def _tc_layer(xp, wn_t, wc_t, bsz, n):
    c = xp.shape[1]
    o = wn_t.shape[1]
    return pl.pallas_call(
        functools.partial(_tc_layer_body, n),
        grid=(bsz,),
        in_specs=[
            pl.BlockSpec((n, c), lambda b: (b, 0)),
            pl.BlockSpec((c, o), lambda b: (0, 0)),
            pl.BlockSpec((c, o), lambda b: (0, 0)),
        ],
        out_specs=[
            pl.BlockSpec((8, n), lambda b: (b, 0)),
            pl.BlockSpec((n, o), lambda b: (b, 0)),
            pl.BlockSpec((n, o), lambda b: (b, 0)),
        ],
        out_shape=[
            jax.ShapeDtypeStruct((bsz * 8, n), jnp.int32),
            jax.ShapeDtypeStruct((bsz * n, o), jnp.float32),
            jax.ShapeDtypeStruct((bsz * n, o), jnp.float32),
        ],
    )(xp, wn_t, wc_t)


def _sc_layer(t, ctr, idx):
    bn, o = t.shape
    n = idx.shape[1]
    nw = 32
    rows_w = bn // nw
    ch = 32
    nch = rows_w // ch
    mesh = plsc.VectorSubcoreMesh(core_axis_name="c", subcore_axis_name="s")

    @functools.partial(
        pl.kernel,
        mesh=mesh,
        out_type=jax.ShapeDtypeStruct((bn, o), jnp.float32),
        scratch_types=[
            pltpu.VMEM((8, rows_w), jnp.int32),
            pltpu.VMEM((ch, o), jnp.float32),
            pltpu.VMEM((ch, o), jnp.float32),
            pltpu.VMEM((ch, o), jnp.float32),
            pltpu.VMEM((ch, o), jnp.float32),
            pltpu.VMEM((ch, o), jnp.float32),
            pltpu.VMEM((ch, o), jnp.float32),
            pltpu.SemaphoreType.DMA,
        ],
    )
    def sc_k(t_hbm, ctr_hbm, idx_hbm, out_hbm, idx_v, r0v, r1v, r2v, r3v,
             ctr_v, out_v, sem):
        wid = lax.axis_index("s") * 2 + lax.axis_index("c")
        base = wid * rows_w
        bi = base // n
        col = base - bi * n
        pltpu.sync_copy(idx_hbm.at[pl.ds(bi * 8, 8), pl.ds(col, rows_w)],
                        idx_v)
        for ci in range(nch):
            p0 = base + ci * ch
            rbufs = (r0v, r1v, r2v, r3v)
            copies = [
                pltpu.async_copy(
                    t_hbm.at[idx_v.at[k, pl.ds(ci * ch, ch)]], rbufs[k], sem)
                for k in range(K)]
            pltpu.sync_copy(ctr_hbm.at[pl.ds(p0, ch)], ctr_v)
            for cpy in copies:
                cpy.wait()

            def body(p, carry):
                for j in range(o // 16):
                    s = pl.ds(j * 16, 16)
                    v = jnp.maximum(
                        jnp.maximum(r0v[p, s], r1v[p, s]),
                        jnp.maximum(r2v[p, s], r3v[p, s]))
                    out_v[p, s] = jnp.maximum(v + ctr_v[p, s], 0.0)
                return carry

            lax.fori_loop(0, ch, body, 0)
            pltpu.sync_copy(out_v, out_hbm.at[pl.ds(p0, ch)])

    return sc_k(t, ctr, idx)


def _tc_final_body(x1_ref, x2_ref, x3_ref, x4_ref, w5_ref, out_ref):
    cat = jnp.concatenate(
        [x1_ref[...][:, :64], x2_ref[...][:, :64], x3_ref[...], x4_ref[...]],
        axis=1)
    r = lax.dot_general(w5_ref[...], cat, (((1,), (1,)), ((), ())),
                        preferred_element_type=jnp.float32)
    out_ref[...] = jnp.tanh(r)[None]


def _tc_final(feats, w5, bsz, n):
    x1, x2, x3, x4 = feats
    specs = [pl.BlockSpec((n, f.shape[1]), lambda b: (b, 0)) for f in feats]
    return pl.pallas_call(
        _tc_final_body,
        grid=(bsz,),
        in_specs=specs + [pl.BlockSpec((512, 512), lambda b: (0, 0))],
        out_specs=pl.BlockSpec((1, 512, n), lambda b: (b, 0, 0)),
        out_shape=jax.ShapeDtypeStruct((bsz, 512, n), jnp.float32),
    )(x1, x2, x3, x4, w5)


def kernel(x, W1, W2, W3, W4, W5):
    bsz, c0, n = x.shape
    xt = jnp.transpose(x, (0, 2, 1)).reshape(bsz * n, c0)
    weights = ((W1, c0), (W2, 64), (W3, 64), (W4, 128))
    hb = bsz // 2
    outs = []
    for h in range(2):
        cur = lax.slice_in_dim(xt, h * hb * n, (h + 1) * hb * n, axis=0)
        feats = []
        for w, cin in weights:
            o = w.shape[0]
            op = max(o, 128)
            cp = cur.shape[1]
            wn_t = jnp.zeros((cp, op), jnp.float32).at[:cin, :o].set(
                jnp.transpose(w[:, :cin]))
            wc_t = jnp.zeros((cp, op), jnp.float32).at[:cin, :o].set(
                jnp.transpose(w[:, cin:]))
            idx, t, ctr = _tc_layer(cur, wn_t, wc_t, hb, n)
            cur = _sc_layer(t, ctr, idx)
            feats.append(cur)
        outs.append(_tc_final(feats, W5, hb, n))
    return jnp.concatenate(outs, axis=0)


# speedup vs baseline: 28.4696x; 1.0686x over previous
import functools

import jax
import jax.numpy as jnp
from jax import lax
from jax.experimental import pallas as pl
from jax.experimental.pallas import tpu as pltpu
from jax.experimental.pallas import tpu_sc as plsc

K = 4


def _tc_layer_body(n, cin, o, op, x_ref, w_ref, idx_ref, t_ref, ctr_ref):
    b = pl.program_id(0)
    xb = x_ref[...][:, :cin]
    sqc = jnp.sum(xb * xb, axis=1, keepdims=True)
    g = lax.dot_general(xb, xb, (((1,), (1,)), ((), ())),
                        preferred_element_type=jnp.float32)
    d = 2.0 * g - sqc
    iotaf = lax.broadcasted_iota(jnp.int32, (n, n), 0).astype(jnp.float32)
    for k in range(K):
        m = jnp.max(d, axis=0, keepdims=True)
        cand = jnp.where(d == m, iotaf, jnp.float32(n))
        ikf = jnp.min(cand, axis=0, keepdims=True)
        idx_ref[k:k + 1, :] = ikf.astype(jnp.int32) + b * n
        if k < K - 1:
            d = jnp.where(iotaf == ikf, -jnp.inf, d)
    idx_ref[K:, :] = jnp.zeros((8 - K, n), jnp.int32)
    w = w_ref[...]
    tn = lax.dot_general(xb, w[:, :cin], (((1,), (1,)), ((), ())),
                         preferred_element_type=jnp.float32)
    tc = lax.dot_general(xb, w[:, cin:], (((1,), (1,)), ((), ())),
                         preferred_element_type=jnp.float32)
    t_ref[:, :o] = tn
    ctr_ref[:, :o] = tc
    if op > o:
        zpad = jnp.zeros((n, op - o), jnp.float32)
        t_ref[:, o:] = zpad
        ctr_ref[:, o:] = zpad


def _tc_layer(xp, w, h0, bsz, n, cin, op):
    cp = xp.shape[1]
    o = w.shape[0]
    return pl.pallas_call(
        functools.partial(_tc_layer_body, n, cin, o, op),
        grid=(bsz,),
        in_specs=[
            pl.BlockSpec((n, cp), lambda b, h0=h0: (b + h0, 0)),
            pl.BlockSpec(w.shape, lambda b: (0, 0)),
        ],
        out_specs=[
            pl.BlockSpec((8, n), lambda b: (b, 0)),
            pl.BlockSpec((n, op), lambda b: (b, 0)),
            pl.BlockSpec((n, op), lambda b: (b, 0)),
        ],
        out_shape=[
            jax.ShapeDtypeStruct((bsz * 8, n), jnp.int32),
            jax.ShapeDtypeStruct((bsz * n, op), jnp.float32),
            jax.ShapeDtypeStruct((bsz * n, op), jnp.float32),
        ],
    )(xp, w)


def _sc_layer(t, ctr, idx):
    bn, o = t.shape
    n = idx.shape[1]
    nw = 32
    rows_w = bn // nw
    ch = 32
    nch = rows_w // ch
    mesh = plsc.VectorSubcoreMesh(core_axis_name="c", subcore_axis_name="s")

    @functools.partial(
        pl.kernel,
        mesh=mesh,
        out_type=jax.ShapeDtypeStruct((bn, o), jnp.float32),
        scratch_types=[
            pltpu.VMEM((8, rows_w), jnp.int32),
            pltpu.VMEM((ch, o), jnp.float32),
            pltpu.VMEM((ch, o), jnp.float32),
            pltpu.VMEM((ch, o), jnp.float32),
            pltpu.VMEM((ch, o), jnp.float32),
            pltpu.VMEM((ch, o), jnp.float32),
            pltpu.VMEM((ch, o), jnp.float32),
            pltpu.SemaphoreType.DMA,
        ],
    )
    def sc_k(t_hbm, ctr_hbm, idx_hbm, out_hbm, idx_v, r0v, r1v, r2v, r3v,
             ctr_v, out_v, sem):
        wid = lax.axis_index("s") * 2 + lax.axis_index("c")
        base = wid * rows_w
        bi = base // n
        col = base - bi * n
        pltpu.sync_copy(idx_hbm.at[pl.ds(bi * 8, 8), pl.ds(col, rows_w)],
                        idx_v)
        for ci in range(nch):
            p0 = base + ci * ch
            rbufs = (r0v, r1v, r2v, r3v)
            copies = [
                pltpu.async_copy(
                    t_hbm.at[idx_v.at[k, pl.ds(ci * ch, ch)]], rbufs[k], sem)
                for k in range(K)]
            pltpu.sync_copy(ctr_hbm.at[pl.ds(p0, ch)], ctr_v)
            for cpy in copies:
                cpy.wait()

            def body(p, carry):
                for j in range(o // 16):
                    s = pl.ds(j * 16, 16)
                    v = jnp.maximum(
                        jnp.maximum(r0v[p, s], r1v[p, s]),
                        jnp.maximum(r2v[p, s], r3v[p, s]))
                    out_v[p, s] = jnp.maximum(v + ctr_v[p, s], 0.0)
                return carry

            lax.fori_loop(0, ch, body, 0)
            pltpu.sync_copy(out_v, out_hbm.at[pl.ds(p0, ch)])

    return sc_k(t, ctr, idx)


def _tc_final_body(x1_ref, x2_ref, x3_ref, x4_ref, w5_ref, out_ref):
    cat = jnp.concatenate(
        [x1_ref[...][:, :64], x2_ref[...][:, :64], x3_ref[...], x4_ref[...]],
        axis=1)
    r = lax.dot_general(w5_ref[...], cat, (((1,), (1,)), ((), ())),
                        preferred_element_type=jnp.float32)
    out_ref[...] = jnp.tanh(r)[None]


def _tc_final(feats, w5, bsz, n):
    x1, x2, x3, x4 = feats
    specs = [pl.BlockSpec((n, f.shape[1]), lambda b: (b, 0)) for f in feats]
    return pl.pallas_call(
        _tc_final_body,
        grid=(bsz,),
        in_specs=specs + [pl.BlockSpec((512, 512), lambda b: (0, 0))],
        out_specs=pl.BlockSpec((1, 512, n), lambda b: (b, 0, 0)),
        out_shape=jax.ShapeDtypeStruct((bsz, 512, n), jnp.float32),
    )(x1, x2, x3, x4, w5)


def kernel(x, W1, W2, W3, W4, W5):
    bsz, c0, n = x.shape
    xt = jnp.transpose(x, (0, 2, 1)).reshape(bsz * n, c0)
    weights = ((W1, c0), (W2, 64), (W3, 64), (W4, 128))
    hb = bsz // 2
    outs = []
    for h in range(2):
        cur, h0 = xt, h * hb
        feats = []
        for w, cin in weights:
            op = max(w.shape[0], 128)
            idx, t, ctr = _tc_layer(cur, w, h0, hb, n, cin, op)
            cur = _sc_layer(t, ctr, idx)
            h0 = 0
            feats.append(cur)
        outs.append(_tc_final(feats, W5, hb, n))
    return jnp.concatenate(outs, axis=0)


# speedup vs baseline: 31.1365x; 1.0937x over previous
import functools

import jax
import jax.numpy as jnp
from jax import lax
from jax.experimental import pallas as pl
from jax.experimental.pallas import tpu as pltpu
from jax.experimental.pallas import tpu_sc as plsc

K = 4


def _tc_layer_body(n, cin, o, op, first, x_ref, w_ref, idx_ref, t_ref,
                   ctr_ref):
    b = pl.program_id(0)
    w = w_ref[...]
    if first:
        xc = x_ref[0]
        g = lax.dot_general(xc, xc, (((0,), (0,)), ((), ())),
                            preferred_element_type=jnp.float32)
        sqc = lax.dot_general(xc * xc, jnp.ones((cin, 1), jnp.float32),
                              (((0,), (0,)), ((), ())),
                              preferred_element_type=jnp.float32)
        tn = lax.dot_general(xc, w[:, :cin], (((0,), (1,)), ((), ())),
                             preferred_element_type=jnp.float32)
        tc = lax.dot_general(xc, w[:, cin:], (((0,), (1,)), ((), ())),
                             preferred_element_type=jnp.float32)
    else:
        xb = x_ref[...][:, :cin]
        sqc = jnp.sum(xb * xb, axis=1, keepdims=True)
        g = lax.dot_general(xb, xb, (((1,), (1,)), ((), ())),
                            preferred_element_type=jnp.float32)
        tn = lax.dot_general(xb, w[:, :cin], (((1,), (1,)), ((), ())),
                             preferred_element_type=jnp.float32)
        tc = lax.dot_general(xb, w[:, cin:], (((1,), (1,)), ((), ())),
                             preferred_element_type=jnp.float32)
    iota0 = lax.broadcasted_iota(jnp.int32, (n, n), 0)
    iotaf = iota0.astype(jnp.float32)
    diag = iota0 == lax.broadcasted_iota(jnp.int32, (n, n), 1)
    d = jnp.where(diag, -jnp.inf, 2.0 * g - sqc)
    idx_ref[0:1, :] = (lax.broadcasted_iota(jnp.int32, (1, n), 1) + b * n)
    for k in range(1, K):
        m = jnp.max(d, axis=0, keepdims=True)
        cand = jnp.where(d == m, iotaf, jnp.float32(n))
        ikf = jnp.min(cand, axis=0, keepdims=True)
        idx_ref[k:k + 1, :] = ikf.astype(jnp.int32) + b * n
        if k < K - 1:
            d = jnp.where(iotaf == ikf, -jnp.inf, d)
    idx_ref[K:, :] = jnp.zeros((8 - K, n), jnp.int32)
    t_ref[:, :o] = tn
    ctr_ref[:, :o] = tc
    if op > o:
        zpad = jnp.zeros((n, op - o), jnp.float32)
        t_ref[:, o:] = zpad
        ctr_ref[:, o:] = zpad


def _tc_layer(xp, w, h0, bsz, n, cin, op):
    o = w.shape[0]
    first = xp.ndim == 3
    if first:
        xspec = pl.BlockSpec((1, cin, n), lambda b, h0=h0: (b + h0, 0, 0))
    else:
        cp = xp.shape[1]
        xspec = pl.BlockSpec((n, cp), lambda b, h0=h0: (b + h0, 0))
    return pl.pallas_call(
        functools.partial(_tc_layer_body, n, cin, o, op, first),
        grid=(bsz,),
        in_specs=[
            xspec,
            pl.BlockSpec(w.shape, lambda b: (0, 0)),
        ],
        out_specs=[
            pl.BlockSpec((8, n), lambda b: (b, 0)),
            pl.BlockSpec((n, op), lambda b: (b, 0)),
            pl.BlockSpec((n, op), lambda b: (b, 0)),
        ],
        out_shape=[
            jax.ShapeDtypeStruct((bsz * 8, n), jnp.int32),
            jax.ShapeDtypeStruct((bsz * n, op), jnp.float32),
            jax.ShapeDtypeStruct((bsz * n, op), jnp.float32),
        ],
    )(xp, w)


def _sc_layer(t, ctr, idx):
    bn, o = t.shape
    n = idx.shape[1]
    nw = 32
    rows_w = bn // nw
    ch = 32
    nch = rows_w // ch
    mesh = plsc.VectorSubcoreMesh(core_axis_name="c", subcore_axis_name="s")

    @functools.partial(
        pl.kernel,
        mesh=mesh,
        out_type=jax.ShapeDtypeStruct((bn, o), jnp.float32),
        scratch_types=[
            pltpu.VMEM((8, rows_w), jnp.int32),
            pltpu.VMEM((ch, o), jnp.float32),
            pltpu.VMEM((ch, o), jnp.float32),
            pltpu.VMEM((ch, o), jnp.float32),
            pltpu.VMEM((ch, o), jnp.float32),
            pltpu.VMEM((ch, o), jnp.float32),
            pltpu.VMEM((ch, o), jnp.float32),
            pltpu.SemaphoreType.DMA,
        ],
    )
    def sc_k(t_hbm, ctr_hbm, idx_hbm, out_hbm, idx_v, r0v, r1v, r2v, r3v,
             ctr_v, out_v, sem):
        wid = lax.axis_index("s") * 2 + lax.axis_index("c")
        base = wid * rows_w
        bi = base // n
        col = base - bi * n
        pltpu.sync_copy(idx_hbm.at[pl.ds(bi * 8, 8), pl.ds(col, rows_w)],
                        idx_v)
        for ci in range(nch):
            p0 = base + ci * ch
            rbufs = (r0v, r1v, r2v, r3v)
            copies = [pltpu.async_copy(t_hbm.at[pl.ds(p0, ch)], r0v, sem)]
            copies += [
                pltpu.async_copy(
                    t_hbm.at[idx_v.at[k, pl.ds(ci * ch, ch)]], rbufs[k], sem)
                for k in range(1, K)]
            pltpu.sync_copy(ctr_hbm.at[pl.ds(p0, ch)], ctr_v)
            for cpy in copies:
                cpy.wait()

            def body(p, carry):
                for j in range(o // 16):
                    s = pl.ds(j * 16, 16)
                    v = jnp.maximum(
                        jnp.maximum(r0v[p, s], r1v[p, s]),
                        jnp.maximum(r2v[p, s], r3v[p, s]))
                    out_v[p, s] = jnp.maximum(v + ctr_v[p, s], 0.0)
                return carry

            lax.fori_loop(0, ch, body, 0)
            pltpu.sync_copy(out_v, out_hbm.at[pl.ds(p0, ch)])

    return sc_k(t, ctr, idx)


def _tc_final_body(x1_ref, x2_ref, x3_ref, x4_ref, w5_ref, out_ref):
    cat = jnp.concatenate(
        [x1_ref[...][:, :64], x2_ref[...][:, :64], x3_ref[...], x4_ref[...]],
        axis=1)
    r = lax.dot_general(w5_ref[...], cat, (((1,), (1,)), ((), ())),
                        preferred_element_type=jnp.float32)
    out_ref[...] = jnp.tanh(r)[None]


def _tc_final(feats, w5, bsz, n):
    x1, x2, x3, x4 = feats
    specs = [pl.BlockSpec((n, f.shape[1]), lambda b: (b, 0)) for f in feats]
    return pl.pallas_call(
        _tc_final_body,
        grid=(bsz,),
        in_specs=specs + [pl.BlockSpec((512, 512), lambda b: (0, 0))],
        out_specs=pl.BlockSpec((1, 512, n), lambda b: (b, 0, 0)),
        out_shape=jax.ShapeDtypeStruct((bsz, 512, n), jnp.float32),
    )(x1, x2, x3, x4, w5)


def kernel(x, W1, W2, W3, W4, W5):
    bsz, c0, n = x.shape
    weights = ((W1, c0), (W2, 64), (W3, 64), (W4, 128))
    hb = bsz // 2
    outs = []
    for h in range(2):
        cur, h0 = x, h * hb
        feats = []
        for w, cin in weights:
            op = max(w.shape[0], 128)
            idx, t, ctr = _tc_layer(cur, w, h0, hb, n, cin, op)
            cur = _sc_layer(t, ctr, idx)
            h0 = 0
            feats.append(cur)
        outs.append(_tc_final(feats, W5, hb, n))
    return jnp.concatenate(outs, axis=0)


# speedup vs baseline: 31.1755x; 1.0013x over previous
import functools

import jax
import jax.numpy as jnp
from jax import lax
from jax.experimental import pallas as pl
from jax.experimental.pallas import tpu as pltpu
from jax.experimental.pallas import tpu_sc as plsc

K = 4


def _tc_layer_body(n, cin, o, op, first, x_ref, w_ref, idx_ref, t_ref,
                   ctr_ref):
    b = pl.program_id(0)
    w = w_ref[...]
    if first:
        xc = x_ref[0]
        g = lax.dot_general(xc, xc, (((0,), (0,)), ((), ())),
                            preferred_element_type=jnp.float32)
        sqc = lax.dot_general(xc * xc, jnp.ones((cin, 1), jnp.float32),
                              (((0,), (0,)), ((), ())),
                              preferred_element_type=jnp.float32)
        tn = lax.dot_general(xc, w[:, :cin], (((0,), (1,)), ((), ())),
                             preferred_element_type=jnp.float32)
        tc = lax.dot_general(xc, w[:, cin:], (((0,), (1,)), ((), ())),
                             preferred_element_type=jnp.float32)
    else:
        xb = x_ref[...][:, :cin]
        sqc = jnp.sum(xb * xb, axis=1, keepdims=True)
        g = lax.dot_general(xb, xb, (((1,), (1,)), ((), ())),
                            preferred_element_type=jnp.float32)
        tn = lax.dot_general(xb, w[:, :cin], (((1,), (1,)), ((), ())),
                             preferred_element_type=jnp.float32)
        tc = lax.dot_general(xb, w[:, cin:], (((1,), (1,)), ((), ())),
                             preferred_element_type=jnp.float32)
    iota0 = lax.broadcasted_iota(jnp.int32, (n, n), 0)
    iotaf = iota0.astype(jnp.float32)
    diag = iota0 == lax.broadcasted_iota(jnp.int32, (n, n), 1)
    d = jnp.where(diag, -jnp.inf, 2.0 * g - sqc)
    idx_ref[0:1, :] = (lax.broadcasted_iota(jnp.int32, (1, n), 1) + b * n)
    for k in range(1, K):
        m = jnp.max(d, axis=0, keepdims=True)
        cand = jnp.where(d == m, iotaf, jnp.float32(n))
        ikf = jnp.min(cand, axis=0, keepdims=True)
        idx_ref[k:k + 1, :] = ikf.astype(jnp.int32) + b * n
        if k < K - 1:
            d = jnp.where(iotaf == ikf, -jnp.inf, d)
    idx_ref[K:, :] = jnp.zeros((8 - K, n), jnp.int32)
    t_ref[:, :o] = tn
    ctr_ref[:, :o] = tc
    if op > o:
        zpad = jnp.zeros((n, op - o), jnp.float32)
        t_ref[:, o:] = zpad
        ctr_ref[:, o:] = zpad


def _tc_layer(xp, w, h0, bsz, n, cin, op):
    o = w.shape[0]
    first = xp.ndim == 3
    if first:
        xspec = pl.BlockSpec((1, cin, n), lambda b, h0=h0: (b + h0, 0, 0))
    else:
        cp = xp.shape[1]
        xspec = pl.BlockSpec((n, cp), lambda b, h0=h0: (b + h0, 0))
    return pl.pallas_call(
        functools.partial(_tc_layer_body, n, cin, o, op, first),
        grid=(bsz,),
        in_specs=[
            xspec,
            pl.BlockSpec(w.shape, lambda b: (0, 0)),
        ],
        out_specs=[
            pl.BlockSpec((8, n), lambda b: (b, 0)),
            pl.BlockSpec((n, op), lambda b: (b, 0)),
            pl.BlockSpec((n, op), lambda b: (b, 0)),
        ],
        out_shape=[
            jax.ShapeDtypeStruct((bsz * 8, n), jnp.int32),
            jax.ShapeDtypeStruct((bsz * n, op), jnp.float32),
            jax.ShapeDtypeStruct((bsz * n, op), jnp.float32),
        ],
    )(xp, w)


def _sc_layer(t, ctr, idx):
    bn, o = t.shape
    n = idx.shape[1]
    nw = 32
    rows_w = bn // nw
    ch = 32
    nch = rows_w // ch
    mesh = plsc.VectorSubcoreMesh(core_axis_name="c", subcore_axis_name="s")

    @functools.partial(
        pl.kernel,
        mesh=mesh,
        out_type=jax.ShapeDtypeStruct((bn, o), jnp.float32),
        scratch_types=[
            pltpu.VMEM((8, rows_w), jnp.int32),
            pltpu.VMEM((ch, o), jnp.float32),
            pltpu.VMEM((ch, o), jnp.float32),
            pltpu.VMEM((ch, o), jnp.float32),
            pltpu.VMEM((ch, o), jnp.float32),
            pltpu.VMEM((ch, o), jnp.float32),
            pltpu.VMEM((ch, o), jnp.float32),
            pltpu.SemaphoreType.DMA,
        ],
    )
    def sc_k(t_hbm, ctr_hbm, idx_hbm, out_hbm, idx_v, r0v, r1v, r2v, r3v,
             ctr_v, out_v, sem):
        wid = lax.axis_index("s") * 2 + lax.axis_index("c")
        base = wid * rows_w
        bi = base // n
        col = base - bi * n
        pltpu.sync_copy(idx_hbm.at[pl.ds(bi * 8, 8), pl.ds(col, rows_w)],
                        idx_v)
        for ci in range(nch):
            p0 = base + ci * ch
            rbufs = (r0v, r1v, r2v, r3v)
            copies = [pltpu.async_copy(t_hbm.at[pl.ds(p0, ch)], r0v, sem)]
            copies += [
                pltpu.async_copy(
                    t_hbm.at[idx_v.at[k, pl.ds(ci * ch, ch)]], rbufs[k], sem)
                for k in range(1, K)]
            pltpu.sync_copy(ctr_hbm.at[pl.ds(p0, ch)], ctr_v)
            for cpy in copies:
                cpy.wait()

            def body(p, carry):
                for j in range(o // 16):
                    s = pl.ds(j * 16, 16)
                    v = jnp.maximum(
                        jnp.maximum(r0v[p, s], r1v[p, s]),
                        jnp.maximum(r2v[p, s], r3v[p, s]))
                    out_v[p, s] = jnp.maximum(v + ctr_v[p, s], 0.0)
                return carry

            lax.fori_loop(0, ch, body, 0)
            pltpu.sync_copy(out_v, out_hbm.at[pl.ds(p0, ch)])

    return sc_k(t, ctr, idx)


def _tc_final_body(x1_ref, x2_ref, x3_ref, x4_ref, w5_ref, out_ref):
    cat = jnp.concatenate(
        [x1_ref[...][:, :64], x2_ref[...][:, :64], x3_ref[...], x4_ref[...]],
        axis=1)
    r = lax.dot_general(w5_ref[...], cat, (((1,), (1,)), ((), ())),
                        preferred_element_type=jnp.float32)
    out_ref[...] = jnp.tanh(r)[None]


def _tc_final(feats, w5, bsz, n):
    x1, x2, x3, x4 = feats
    specs = [pl.BlockSpec((n, f.shape[1]), lambda b: (b, 0)) for f in feats]
    return pl.pallas_call(
        _tc_final_body,
        grid=(bsz,),
        in_specs=specs + [pl.BlockSpec((512, 512), lambda b: (0, 0))],
        out_specs=pl.BlockSpec((1, 512, n), lambda b: (b, 0, 0)),
        out_shape=jax.ShapeDtypeStruct((bsz, 512, n), jnp.float32),
    )(x1, x2, x3, x4, w5)


def kernel(x, W1, W2, W3, W4, W5):
    bsz, c0, n = x.shape
    weights = ((W1, c0), (W2, 64), (W3, 64), (W4, 128))
    hb = bsz // 2
    outs = []
    xt = jnp.transpose(x, (0, 2, 1)).reshape(bsz * n, c0)
    for h in range(2):
        cur, h0 = xt, h * hb
        feats = []
        for w, cin in weights:
            op = max(w.shape[0], 128)
            idx, t, ctr = _tc_layer(cur, w, h0, hb, n, cin, op)
            cur = _sc_layer(t, ctr, idx)
            h0 = 0
            feats.append(cur)
        outs.append(_tc_final(feats, W5, hb, n))
    return jnp.concatenate(outs, axis=0)
